# baseline (speedup 1.0000x reference)
import functools

import jax
import jax.numpy as jnp
from jax import lax
from jax.experimental import pallas as pl
from jax.experimental.pallas import tpu as pltpu
from jax.experimental.pallas import tpu_sc as plsc

NC = 2
NS = 16
NW = NC * NS
LANES = 16
CH = 128
KB = 8


def _sc_partials_kernel(N, DH, NCH, ACC_ROWS, RPW):
    mesh = plsc.VectorSubcoreMesh(core_axis_name="c", subcore_axis_name="s")

    @functools.partial(
        pl.kernel,
        mesh=mesh,
        out_type=jax.ShapeDtypeStruct((NC, 2, ACC_ROWS, DH), jnp.float32),
        scratch_types=[
            pltpu.VMEM((NCH, CH), jnp.int32),
            pltpu.VMEM((NCH, CH), jnp.int32),
            [pltpu.VMEM((CH, DH), jnp.float32) for _ in range(KB)],
            pltpu.VMEM_SHARED((ACC_ROWS, DH), jnp.float32),
            pltpu.SemaphoreType.DMA,
            pltpu.SemaphoreType.DMA,
        ],
        compiler_params=pltpu.CompilerParams(use_tc_tiling_on_sc=False),
    )
    def k(x0_hbm, x1_hbm, idx0_hbm, idx1_hbm, out_hbm, idx0_v, idx1_v,
          rows, acc_sh, gsem, ssem):
        rows_v = rows[0]
        zeros_v = rows[1]
        c = lax.axis_index("c")
        s = lax.axis_index("s")
        w = c * NS + s

        pltpu.sync_copy(idx0_hbm.at[w], idx0_v)
        pltpu.sync_copy(idx1_hbm.at[w], idx1_v)

        zvec = jnp.zeros((LANES,), jnp.float32)
        base = s * RPW
        full = RPW // CH
        rem = RPW - full * CH

        for h, xh in ((0, x0_hbm), (1, x1_hbm)):
            def zrow(i, _):
                for j in range(DH // LANES):
                    zeros_v[i, pl.ds(j * LANES, LANES)] = zvec
                return 0

            lax.fori_loop(0, CH, zrow, 0)

            for t in range(full):
                pltpu.sync_copy(zeros_v, acc_sh.at[pl.ds(base + t * CH, CH)])
            if rem:
                pltpu.sync_copy(zeros_v.at[pl.ds(0, rem)],
                                acc_sh.at[pl.ds(base + full * CH, rem)])
            plsc.subcore_barrier()

            def body(su, _):
                j0 = su * KB
                gds = [pltpu.async_copy(xh.at[idx0_v.at[j0 + b]], rows[b],
                                        gsem) for b in range(KB)]
                for d in gds:
                    d.wait()
                sds = [pltpu.async_copy(rows[b],
                                        acc_sh.at[idx1_v.at[j0 + b]],
                                        ssem, add=True) for b in range(KB)]
                for d in sds:
                    d.wait()
                return 0

            lax.fori_loop(0, NCH // KB, body, 0)
            plsc.subcore_barrier()

            for t in range(full):
                pltpu.sync_copy(acc_sh.at[pl.ds(base + t * CH, CH)], rows_v)
                pltpu.sync_copy(rows_v,
                                out_hbm.at[c, h, pl.ds(base + t * CH, CH)])
            if rem:
                pltpu.sync_copy(acc_sh.at[pl.ds(base + full * CH, rem)],
                                rows_v.at[pl.ds(0, rem)])
                pltpu.sync_copy(rows_v.at[pl.ds(0, rem)],
                                out_hbm.at[c, h, pl.ds(base + full * CH, rem)])

    return k


def _add_body(a_ref, b_ref, o_ref):
    o_ref[...] = jnp.concatenate(
        [a_ref[0, 0] + b_ref[0, 0], a_ref[0, 1] + b_ref[0, 1]], axis=1)


def kernel(x, index):
    N, D = x.shape
    DH = D // 2
    E = index.shape[1]
    assert E % NW == 0 and D % 2 == 0
    EW = E // NW
    NCH = -(-EW // (CH * KB)) * KB
    EP = NCH * CH
    ACC_ROWS = -(-(N + 1) // (NS * 8)) * (NS * 8)
    RPW = ACC_ROWS // NS

    idx = index.astype(jnp.int32)
    i0 = jnp.pad(idx[0].reshape(NW, EW), ((0, 0), (0, EP - EW)))
    i1 = jnp.pad(idx[1].reshape(NW, EW), ((0, 0), (0, EP - EW)),
                 constant_values=N)
    i0 = i0.reshape(NW, NCH, CH)
    i1 = i1.reshape(NW, NCH, CH)

    x0 = x[:, :DH]
    x1 = x[:, DH:]
    partials = _sc_partials_kernel(N, DH, NCH, ACC_ROWS, RPW)(x0, x1, i0, i1)

    BR = 2000
    out = pl.pallas_call(
        _add_body,
        out_shape=jax.ShapeDtypeStruct((N, D), jnp.float32),
        grid=(N // BR,),
        in_specs=[pl.BlockSpec((1, 2, BR, DH), lambda i: (0, 0, i, 0)),
                  pl.BlockSpec((1, 2, BR, DH), lambda i: (1, 0, i, 0))],
        out_specs=pl.BlockSpec((BR, D), lambda i: (i, 0)),
    )(partials, partials)
    return out


# speedup vs baseline: 1.9130x; 1.9130x over previous
import functools

import jax
import jax.numpy as jnp
from jax import lax
from jax.experimental import pallas as pl
from jax.experimental.pallas import tpu as pltpu
from jax.experimental.pallas import tpu_sc as plsc

NC = 2
NS = 16
NW = NC * NS
LANES = 16
CH = 128
KB = 3


def _sc_partials_kernel(N, DH, NCH, ACC_ROWS, RPW):
    mesh = plsc.VectorSubcoreMesh(core_axis_name="c", subcore_axis_name="s")

    @functools.partial(
        pl.kernel,
        mesh=mesh,
        out_type=jax.ShapeDtypeStruct((NC, 2, ACC_ROWS, DH), jnp.float32),
        scratch_types=[
            pltpu.VMEM((NCH, CH), jnp.int32),
            pltpu.VMEM((NCH, CH), jnp.int32),
            [pltpu.VMEM((CH, DH), jnp.float32) for _ in range(KB)],
            pltpu.VMEM_SHARED((ACC_ROWS, DH), jnp.float32),
            pltpu.VMEM_SHARED((ACC_ROWS, DH), jnp.float32),
            pltpu.SemaphoreType.DMA,
            pltpu.SemaphoreType.DMA,
        ],
        compiler_params=pltpu.CompilerParams(use_tc_tiling_on_sc=False),
    )
    def k(x0_hbm, x1_hbm, idx0_hbm, idx1_hbm, out_hbm, idx0_v, idx1_v,
          rows, acc_sh, x_sh, gsem, ssem):
        rows_v = rows[0]
        zeros_v = rows[1]
        c = lax.axis_index("c")
        s = lax.axis_index("s")
        w = c * NS + s

        pltpu.sync_copy(idx0_hbm.at[w], idx0_v)
        pltpu.sync_copy(idx1_hbm.at[w], idx1_v)

        zvec = jnp.zeros((LANES,), jnp.float32)
        base = s * RPW
        full = RPW // CH
        rem = RPW - full * CH

        for h, xh in ((0, x0_hbm), (1, x1_hbm)):
            def zrow(i, _):
                for j in range(DH // LANES):
                    zeros_v[i, pl.ds(j * LANES, LANES)] = zvec
                return 0

            lax.fori_loop(0, CH, zrow, 0)

            for t in range(full):
                pltpu.sync_copy(zeros_v, acc_sh.at[pl.ds(base + t * CH, CH)])
            if rem:
                pltpu.sync_copy(zeros_v.at[pl.ds(0, rem)],
                                acc_sh.at[pl.ds(base + full * CH, rem)])

            for t in range(full):
                pltpu.sync_copy(xh.at[pl.ds(base + t * CH, CH)], rows_v)
                pltpu.sync_copy(rows_v, x_sh.at[pl.ds(base + t * CH, CH)])
            if rem:
                pltpu.sync_copy(xh.at[pl.ds(base + full * CH, rem)],
                                rows_v.at[pl.ds(0, rem)])
                pltpu.sync_copy(rows_v.at[pl.ds(0, rem)],
                                x_sh.at[pl.ds(base + full * CH, rem)])
            plsc.subcore_barrier()

            def body(su, _):
                j0 = su * KB
                gds = [pltpu.async_copy(x_sh.at[idx0_v.at[j0 + b]], rows[b],
                                        gsem) for b in range(KB)]
                for d in gds:
                    d.wait()
                sds = [pltpu.async_copy(rows[b],
                                        acc_sh.at[idx1_v.at[j0 + b]],
                                        ssem, add=True) for b in range(KB)]
                for d in sds:
                    d.wait()
                return 0

            lax.fori_loop(0, NCH // KB, body, 0)
            plsc.subcore_barrier()

            for t in range(full):
                pltpu.sync_copy(acc_sh.at[pl.ds(base + t * CH, CH)], rows_v)
                pltpu.sync_copy(rows_v,
                                out_hbm.at[c, h, pl.ds(base + t * CH, CH)])
            if rem:
                pltpu.sync_copy(acc_sh.at[pl.ds(base + full * CH, rem)],
                                rows_v.at[pl.ds(0, rem)])
                pltpu.sync_copy(rows_v.at[pl.ds(0, rem)],
                                out_hbm.at[c, h, pl.ds(base + full * CH, rem)])

    return k


def _add_body(a_ref, b_ref, o_ref):
    o_ref[...] = jnp.concatenate(
        [a_ref[0, 0] + b_ref[0, 0], a_ref[0, 1] + b_ref[0, 1]], axis=1)


def kernel(x, index):
    N, D = x.shape
    DH = D // 2
    E = index.shape[1]
    assert E % NW == 0 and D % 2 == 0
    EW = E // NW
    NCH = -(-EW // (CH * KB)) * KB
    EP = NCH * CH
    ACC_ROWS = -(-(N + 1) // (NS * 8)) * (NS * 8)
    RPW = ACC_ROWS // NS

    idx = index.astype(jnp.int32)
    i0 = jnp.pad(idx[0].reshape(NW, EW), ((0, 0), (0, EP - EW)))
    i1 = jnp.pad(idx[1].reshape(NW, EW), ((0, 0), (0, EP - EW)),
                 constant_values=N)
    i0 = i0.reshape(NW, NCH, CH)
    i1 = i1.reshape(NW, NCH, CH)

    xp = jnp.pad(x, ((0, ACC_ROWS - N), (0, 0)))
    x0 = xp[:, :DH]
    x1 = xp[:, DH:]
    partials = _sc_partials_kernel(N, DH, NCH, ACC_ROWS, RPW)(x0, x1, i0, i1)

    BR = 2000
    out = pl.pallas_call(
        _add_body,
        out_shape=jax.ShapeDtypeStruct((N, D), jnp.float32),
        grid=(N // BR,),
        in_specs=[pl.BlockSpec((1, 2, BR, DH), lambda i: (0, 0, i, 0)),
                  pl.BlockSpec((1, 2, BR, DH), lambda i: (1, 0, i, 0))],
        out_specs=pl.BlockSpec((BR, D), lambda i: (i, 0)),
    )(partials, partials)
    return out


# speedup vs baseline: 2.0133x; 1.0524x over previous
import functools

import jax
import jax.numpy as jnp
from jax import lax
from jax.experimental import pallas as pl
from jax.experimental.pallas import tpu as pltpu
from jax.experimental.pallas import tpu_sc as plsc

NC = 2
NS = 16
NW = NC * NS
LANES = 16
CH = 128
KB = 3


def _sc_partials_kernel(N, DH, NCH, ACC_ROWS, RPW):
    mesh = plsc.VectorSubcoreMesh(core_axis_name="c", subcore_axis_name="s")

    @functools.partial(
        pl.kernel,
        mesh=mesh,
        out_type=jax.ShapeDtypeStruct((NC, 2, ACC_ROWS, DH), jnp.float32),
        scratch_types=[
            pltpu.VMEM((NCH, CH), jnp.int32),
            pltpu.VMEM((NCH, CH), jnp.int32),
            [pltpu.VMEM((CH, DH), jnp.float32) for _ in range(KB)],
            pltpu.VMEM_SHARED((ACC_ROWS, DH), jnp.float32),
            pltpu.VMEM_SHARED((ACC_ROWS, DH), jnp.float32),
            pltpu.SemaphoreType.DMA,
            pltpu.SemaphoreType.DMA,
        ],
        compiler_params=pltpu.CompilerParams(use_tc_tiling_on_sc=False),
    )
    def k(x_hbm, idx0_hbm, idx1_hbm, out_hbm, idx0_v, idx1_v,
          rows, acc_sh, x_sh, gsem, ssem):
        rows_v = rows[0]
        zeros_v = rows[1]
        c = lax.axis_index("c")
        s = lax.axis_index("s")
        w = c * NS + s

        pltpu.sync_copy(idx0_hbm.at[w], idx0_v)
        pltpu.sync_copy(idx1_hbm.at[w], idx1_v)

        zvec = jnp.zeros((LANES,), jnp.float32)
        base = s * RPW
        full = RPW // CH
        rem = RPW - full * CH
        last_rows = N - (NS - 1) * RPW
        lfull = last_rows // CH
        lrem = last_rows - lfull * CH

        for h in (0, 1):
            def zrow(i, _):
                for j in range(DH // LANES):
                    zeros_v[i, pl.ds(j * LANES, LANES)] = zvec
                return 0

            lax.fori_loop(0, CH, zrow, 0)

            for t in range(full):
                pltpu.sync_copy(zeros_v, acc_sh.at[pl.ds(base + t * CH, CH)])
            if rem:
                pltpu.sync_copy(zeros_v.at[pl.ds(0, rem)],
                                acc_sh.at[pl.ds(base + full * CH, rem)])

            csl = pl.ds(h * DH, DH)

            @pl.when(s < NS - 1)
            def _stage_full():
                for t in range(full):
                    rsl = pl.ds(base + t * CH, CH)
                    pltpu.sync_copy(x_hbm.at[rsl, csl], rows_v)
                    pltpu.sync_copy(rows_v, x_sh.at[rsl])
                if rem:
                    rsl = pl.ds(base + full * CH, rem)
                    pltpu.sync_copy(x_hbm.at[rsl, csl],
                                    rows_v.at[pl.ds(0, rem)])
                    pltpu.sync_copy(rows_v.at[pl.ds(0, rem)], x_sh.at[rsl])

            @pl.when(s == NS - 1)
            def _stage_last():
                for t in range(lfull):
                    rsl = pl.ds(base + t * CH, CH)
                    pltpu.sync_copy(x_hbm.at[rsl, csl], rows_v)
                    pltpu.sync_copy(rows_v, x_sh.at[rsl])
                if lrem:
                    rsl = pl.ds(base + lfull * CH, lrem)
                    pltpu.sync_copy(x_hbm.at[rsl, csl],
                                    rows_v.at[pl.ds(0, lrem)])
                    pltpu.sync_copy(rows_v.at[pl.ds(0, lrem)], x_sh.at[rsl])

            plsc.subcore_barrier()

            def body(su, _):
                j0 = su * KB
                gds = [pltpu.async_copy(x_sh.at[idx0_v.at[j0 + b]], rows[b],
                                        gsem) for b in range(KB)]
                for d in gds:
                    d.wait()
                sds = [pltpu.async_copy(rows[b],
                                        acc_sh.at[idx1_v.at[j0 + b]],
                                        ssem, add=True) for b in range(KB)]
                for d in sds:
                    d.wait()
                return 0

            lax.fori_loop(0, NCH // KB, body, 0)
            plsc.subcore_barrier()

            for t in range(full):
                pltpu.sync_copy(acc_sh.at[pl.ds(base + t * CH, CH)], rows_v)
                pltpu.sync_copy(rows_v,
                                out_hbm.at[c, h, pl.ds(base + t * CH, CH)])
            if rem:
                pltpu.sync_copy(acc_sh.at[pl.ds(base + full * CH, rem)],
                                rows_v.at[pl.ds(0, rem)])
                pltpu.sync_copy(rows_v.at[pl.ds(0, rem)],
                                out_hbm.at[c, h, pl.ds(base + full * CH, rem)])

    return k


def _add_body(a_ref, b_ref, o_ref):
    o_ref[...] = jnp.concatenate(
        [a_ref[0, 0] + b_ref[0, 0], a_ref[0, 1] + b_ref[0, 1]], axis=1)


def kernel(x, index):
    N, D = x.shape
    DH = D // 2
    E = index.shape[1]
    assert E % NW == 0 and D % 2 == 0
    EW = E // NW
    NCH = -(-EW // (CH * KB)) * KB
    EP = NCH * CH
    ACC_ROWS = -(-(N + 1) // (NS * 8)) * (NS * 8)
    RPW = ACC_ROWS // NS

    idx = index.astype(jnp.int32)
    i0 = jnp.pad(idx[0].reshape(NW, EW), ((0, 0), (0, EP - EW)))
    i1 = jnp.pad(idx[1].reshape(NW, EW), ((0, 0), (0, EP - EW)),
                 constant_values=N)
    i0 = i0.reshape(NW, NCH, CH)
    i1 = i1.reshape(NW, NCH, CH)

    partials = _sc_partials_kernel(N, DH, NCH, ACC_ROWS, RPW)(x, i0, i1)

    BR = 2000
    out = pl.pallas_call(
        _add_body,
        out_shape=jax.ShapeDtypeStruct((N, D), jnp.float32),
        grid=(N // BR,),
        in_specs=[pl.BlockSpec((1, 2, BR, DH), lambda i: (0, 0, i, 0)),
                  pl.BlockSpec((1, 2, BR, DH), lambda i: (1, 0, i, 0))],
        out_specs=pl.BlockSpec((BR, D), lambda i: (i, 0)),
    )(partials, partials)
    return out


# speedup vs baseline: 2.1830x; 1.0843x over previous
import functools

import jax
import jax.numpy as jnp
from jax import lax
from jax.experimental import pallas as pl
from jax.experimental.pallas import tpu as pltpu
from jax.experimental.pallas import tpu_sc as plsc

NC = 2
NS = 16
NW = NC * NS
LANES = 16
CH = 128
KB = 3


def _sc_partials_kernel(N, DH, NCH, ACC_ROWS, RPW):
    mesh = plsc.VectorSubcoreMesh(core_axis_name="c", subcore_axis_name="s")

    @functools.partial(
        pl.kernel,
        mesh=mesh,
        out_type=jax.ShapeDtypeStruct((NC, ACC_ROWS, 2 * DH), jnp.float32),
        scratch_types=[
            pltpu.VMEM((NCH, CH), jnp.int32),
            pltpu.VMEM((NCH, CH), jnp.int32),
            [pltpu.VMEM((CH, DH), jnp.float32) for _ in range(KB)],
            pltpu.VMEM_SHARED((ACC_ROWS, DH), jnp.float32),
            pltpu.VMEM_SHARED((ACC_ROWS, DH), jnp.float32),
            pltpu.SemaphoreType.DMA,
            pltpu.SemaphoreType.DMA,
        ],
        compiler_params=pltpu.CompilerParams(use_tc_tiling_on_sc=False),
    )
    def k(x_hbm, idx0_hbm, idx1_hbm, out_hbm, idx0_v, idx1_v,
          rows, acc_sh, x_sh, gsem, ssem):
        rows_v = rows[0]
        zeros_v = rows[1]
        c = lax.axis_index("c")
        s = lax.axis_index("s")
        w = c * NS + s

        pltpu.sync_copy(idx0_hbm.at[w], idx0_v)
        pltpu.sync_copy(idx1_hbm.at[w], idx1_v)

        zvec = jnp.zeros((LANES,), jnp.float32)
        base = s * RPW
        full = RPW // CH
        rem = RPW - full * CH
        last_rows = N - (NS - 1) * RPW
        lfull = last_rows // CH
        lrem = last_rows - lfull * CH

        for h in (0, 1):
            def zrow(i, _):
                for j in range(DH // LANES):
                    zeros_v[i, pl.ds(j * LANES, LANES)] = zvec
                return 0

            lax.fori_loop(0, CH, zrow, 0)

            for t in range(full):
                pltpu.sync_copy(zeros_v, acc_sh.at[pl.ds(base + t * CH, CH)])
            if rem:
                pltpu.sync_copy(zeros_v.at[pl.ds(0, rem)],
                                acc_sh.at[pl.ds(base + full * CH, rem)])

            csl = pl.ds(h * DH, DH)

            @pl.when(s < NS - 1)
            def _stage_full():
                for t in range(full):
                    rsl = pl.ds(base + t * CH, CH)
                    pltpu.sync_copy(x_hbm.at[rsl, csl], rows_v)
                    pltpu.sync_copy(rows_v, x_sh.at[rsl])
                if rem:
                    rsl = pl.ds(base + full * CH, rem)
                    pltpu.sync_copy(x_hbm.at[rsl, csl],
                                    rows_v.at[pl.ds(0, rem)])
                    pltpu.sync_copy(rows_v.at[pl.ds(0, rem)], x_sh.at[rsl])

            @pl.when(s == NS - 1)
            def _stage_last():
                for t in range(lfull):
                    rsl = pl.ds(base + t * CH, CH)
                    pltpu.sync_copy(x_hbm.at[rsl, csl], rows_v)
                    pltpu.sync_copy(rows_v, x_sh.at[rsl])
                if lrem:
                    rsl = pl.ds(base + lfull * CH, lrem)
                    pltpu.sync_copy(x_hbm.at[rsl, csl],
                                    rows_v.at[pl.ds(0, lrem)])
                    pltpu.sync_copy(rows_v.at[pl.ds(0, lrem)], x_sh.at[rsl])

            plsc.subcore_barrier()

            def body(su, _):
                j0 = su * KB
                gds = [pltpu.async_copy(x_sh.at[idx0_v.at[j0 + b]], rows[b],
                                        gsem) for b in range(KB)]
                for d in gds:
                    d.wait()
                sds = [pltpu.async_copy(rows[b],
                                        acc_sh.at[idx1_v.at[j0 + b]],
                                        ssem, add=True) for b in range(KB)]
                for d in sds:
                    d.wait()
                return 0

            lax.fori_loop(0, NCH // KB, body, 0)
            plsc.subcore_barrier()

            for t in range(full):
                pltpu.sync_copy(acc_sh.at[pl.ds(base + t * CH, CH)], rows_v)
                pltpu.sync_copy(rows_v,
                                out_hbm.at[c, pl.ds(base + t * CH, CH), csl])
            if rem:
                pltpu.sync_copy(acc_sh.at[pl.ds(base + full * CH, rem)],
                                rows_v.at[pl.ds(0, rem)])
                pltpu.sync_copy(rows_v.at[pl.ds(0, rem)],
                                out_hbm.at[c, pl.ds(base + full * CH, rem),
                                           csl])

    return k


def _add_body(a_ref, b_ref, o_ref):
    o_ref[...] = a_ref[0] + b_ref[0]


def kernel(x, index):
    N, D = x.shape
    DH = D // 2
    E = index.shape[1]
    assert E % NW == 0 and D % 2 == 0
    EW = E // NW
    NCH = -(-EW // (CH * KB)) * KB
    EP = NCH * CH
    ACC_ROWS = -(-(N + 1) // (NS * 8)) * (NS * 8)
    RPW = ACC_ROWS // NS

    idx = index.astype(jnp.int32)
    i0 = jnp.pad(idx[0].reshape(NW, EW), ((0, 0), (0, EP - EW)))
    i1 = jnp.pad(idx[1].reshape(NW, EW), ((0, 0), (0, EP - EW)),
                 constant_values=N)
    i0 = i0.reshape(NW, NCH, CH)
    i1 = i1.reshape(NW, NCH, CH)

    partials = _sc_partials_kernel(N, DH, NCH, ACC_ROWS, RPW)(x, i0, i1)

    BR = 2000
    out = pl.pallas_call(
        _add_body,
        out_shape=jax.ShapeDtypeStruct((N, D), jnp.float32),
        grid=(N // BR,),
        in_specs=[pl.BlockSpec((1, BR, D), lambda i: (0, i, 0)),
                  pl.BlockSpec((1, BR, D), lambda i: (1, i, 0))],
        out_specs=pl.BlockSpec((BR, D), lambda i: (i, 0)),
    )(partials, partials)
    return out


# speedup vs baseline: 2.2952x; 1.0514x over previous
import functools

import jax
import jax.numpy as jnp
from jax import lax
from jax.experimental import pallas as pl
from jax.experimental.pallas import tpu as pltpu
from jax.experimental.pallas import tpu_sc as plsc

NC = 2
NS = 16
NW = NC * NS
LANES = 16
CH = 128
KB = 3


def _sc_partials_kernel(N, DH, NCH, ACC_ROWS, RPW, EW):
    mesh = plsc.VectorSubcoreMesh(core_axis_name="c", subcore_axis_name="s")

    @functools.partial(
        pl.kernel,
        mesh=mesh,
        out_type=jax.ShapeDtypeStruct((NC, ACC_ROWS, 2 * DH), jnp.float32),
        scratch_types=[
            pltpu.VMEM((NCH * CH,), jnp.int32),
            pltpu.VMEM((NCH, CH), jnp.int32),
            [pltpu.VMEM((CH, DH), jnp.float32) for _ in range(KB)],
            pltpu.VMEM_SHARED((ACC_ROWS, DH), jnp.float32),
            pltpu.VMEM_SHARED((ACC_ROWS, DH), jnp.float32),
            pltpu.SemaphoreType.DMA,
            pltpu.SemaphoreType.DMA,
        ],
        compiler_params=pltpu.CompilerParams(use_tc_tiling_on_sc=False),
    )
    def k(x_hbm, idx_hbm, out_hbm, idx0_v, idx1_v,
          rows, acc_sh, x_sh, gsem, ssem):
        rows_v = rows[0]
        zeros_v = rows[1]
        c = lax.axis_index("c")
        s = lax.axis_index("s")
        w = c * NS + s

        off0 = w * EW
        efull = EW // CH
        etail = EW - efull * CH
        pltpu.sync_copy(idx_hbm.at[0, pl.ds(off0, EW)],
                        idx0_v.at[pl.ds(0, EW)])

        def stg(j, _):
            pltpu.async_copy(idx_hbm.at[1, pl.ds(off0 + j * CH, CH)],
                             idx1_v.at[j], ssem)
            return 0

        lax.fori_loop(0, efull, stg, 0)
        if etail:
            pltpu.sync_copy(idx_hbm.at[1, pl.ds(off0 + efull * CH, etail)],
                            idx1_v.at[efull, pl.ds(0, etail)])

        zi = jnp.zeros((LANES,), jnp.int32)
        gi = jnp.full((LANES,), N, jnp.int32)
        for p in range(EW, NCH * CH, LANES):
            idx0_v[pl.ds(p, LANES)] = zi
            idx1_v[p // CH, pl.ds(p % CH, LANES)] = gi

        def drn(j, _):
            pltpu.make_async_copy(idx_hbm.at[1, pl.ds(off0 + j * CH, CH)],
                                  idx1_v.at[j], ssem).wait()
            return 0

        lax.fori_loop(0, efull, drn, 0)

        zvec = jnp.zeros((LANES,), jnp.float32)
        base = s * RPW
        full = RPW // CH
        rem = RPW - full * CH
        last_rows = N - (NS - 1) * RPW
        lfull = last_rows // CH
        lrem = last_rows - lfull * CH

        for h in (0, 1):
            def zrow(i, _):
                for j in range(DH // LANES):
                    zeros_v[i, pl.ds(j * LANES, LANES)] = zvec
                return 0

            lax.fori_loop(0, CH, zrow, 0)

            for t in range(full):
                pltpu.sync_copy(zeros_v, acc_sh.at[pl.ds(base + t * CH, CH)])
            if rem:
                pltpu.sync_copy(zeros_v.at[pl.ds(0, rem)],
                                acc_sh.at[pl.ds(base + full * CH, rem)])

            csl = pl.ds(h * DH, DH)

            @pl.when(s < NS - 1)
            def _stage_full():
                for t in range(full):
                    rsl = pl.ds(base + t * CH, CH)
                    pltpu.sync_copy(x_hbm.at[rsl, csl], rows_v)
                    pltpu.sync_copy(rows_v, x_sh.at[rsl])
                if rem:
                    rsl = pl.ds(base + full * CH, rem)
                    pltpu.sync_copy(x_hbm.at[rsl, csl],
                                    rows_v.at[pl.ds(0, rem)])
                    pltpu.sync_copy(rows_v.at[pl.ds(0, rem)], x_sh.at[rsl])

            @pl.when(s == NS - 1)
            def _stage_last():
                for t in range(lfull):
                    rsl = pl.ds(base + t * CH, CH)
                    pltpu.sync_copy(x_hbm.at[rsl, csl], rows_v)
                    pltpu.sync_copy(rows_v, x_sh.at[rsl])
                if lrem:
                    rsl = pl.ds(base + lfull * CH, lrem)
                    pltpu.sync_copy(x_hbm.at[rsl, csl],
                                    rows_v.at[pl.ds(0, lrem)])
                    pltpu.sync_copy(rows_v.at[pl.ds(0, lrem)], x_sh.at[rsl])

            plsc.subcore_barrier()

            def body(su, _):
                j0 = su * KB
                gds = [pltpu.async_copy(
                    x_sh.at[idx0_v.at[pl.ds((j0 + b) * CH, CH)]], rows[b],
                    gsem) for b in range(KB)]
                for d in gds:
                    d.wait()
                sds = [pltpu.async_copy(rows[b],
                                        acc_sh.at[idx1_v.at[j0 + b]],
                                        ssem, add=True) for b in range(KB)]
                for d in sds:
                    d.wait()
                return 0

            lax.fori_loop(0, NCH // KB, body, 0)
            plsc.subcore_barrier()

            for t in range(full):
                pltpu.sync_copy(acc_sh.at[pl.ds(base + t * CH, CH)], rows_v)
                pltpu.sync_copy(rows_v,
                                out_hbm.at[c, pl.ds(base + t * CH, CH), csl])
            if rem:
                pltpu.sync_copy(acc_sh.at[pl.ds(base + full * CH, rem)],
                                rows_v.at[pl.ds(0, rem)])
                pltpu.sync_copy(rows_v.at[pl.ds(0, rem)],
                                out_hbm.at[c, pl.ds(base + full * CH, rem),
                                           csl])

    return k


def _add_body(a_ref, b_ref, o_ref):
    o_ref[...] = a_ref[0] + b_ref[0]


def kernel(x, index):
    N, D = x.shape
    DH = D // 2
    E = index.shape[1]
    assert E % NW == 0 and D % 2 == 0
    EW = E // NW
    NCH = -(-EW // (CH * KB)) * KB
    EP = NCH * CH
    ACC_ROWS = -(-(N + 1) // (NS * 8)) * (NS * 8)
    RPW = ACC_ROWS // NS

    idx = index.astype(jnp.int32)
    partials = _sc_partials_kernel(N, DH, NCH, ACC_ROWS, RPW, EW)(x, idx)

    BR = 2000
    out = pl.pallas_call(
        _add_body,
        out_shape=jax.ShapeDtypeStruct((N, D), jnp.float32),
        grid=(N // BR,),
        in_specs=[pl.BlockSpec((1, BR, D), lambda i: (0, i, 0)),
                  pl.BlockSpec((1, BR, D), lambda i: (1, i, 0))],
        out_specs=pl.BlockSpec((BR, D), lambda i: (i, 0)),
    )(partials, partials)
    return out
